# Initial kernel scaffold; baseline (speedup 1.0000x reference)
import jax
import jax.numpy as jnp
from jax.experimental import pallas as pl


def kernel(x, h_src, c_src, child_idx, t, W_iou, U_iou, b_iou, U_f_w, U_f_b, W_iou_s, U_iou_s, b_iou_s, U_f_s_w, U_f_s_b):
    raise NotImplementedError("write your pallas kernel here")


# speedup vs baseline: 2.3992x; 2.3992x over previous
import functools

import jax
import jax.numpy as jnp
from jax import lax
from jax.experimental import pallas as pl
from jax.experimental.pallas import tpu as pltpu
from jax.experimental.pallas import tpu_sc as plsc

N = 100000
X = 128
H = 128

_NC = 2
_NS = 16
_NW = _NC * _NS
_CHUNK = 128
_CPW = 49
_PAD2N = _NW * _CPW * _CHUNK


def _gather_body(h_hbm, c_hbm, idx_hbm, hout_hbm, cout_hbm,
                 idx_v, h_v, c_v, sem):
    wid = lax.axis_index("s") * _NC + lax.axis_index("c")

    def chunk(j, carry):
        base = (wid * _CPW + j) * _CHUNK
        pltpu.sync_copy(idx_hbm.at[pl.ds(base, _CHUNK)], idx_v)
        pltpu.async_copy(h_hbm.at[idx_v], h_v, sem).wait()
        pltpu.async_copy(c_hbm.at[idx_v], c_v, sem).wait()
        pltpu.sync_copy(h_v, hout_hbm.at[pl.ds(base, _CHUNK)])
        pltpu.sync_copy(c_v, cout_hbm.at[pl.ds(base, _CHUNK)])
        return carry

    lax.fori_loop(0, _CPW, chunk, 0)


@functools.cache
def _make_sc_gather():
    return functools.partial(
        pl.kernel,
        mesh=plsc.VectorSubcoreMesh(core_axis_name="c", subcore_axis_name="s"),
        out_type=(
            jax.ShapeDtypeStruct((_PAD2N, H), jnp.float32),
            jax.ShapeDtypeStruct((_PAD2N, H), jnp.float32),
        ),
        scratch_types=[
            pltpu.VMEM((_CHUNK,), jnp.int32),
            pltpu.VMEM((_CHUNK, H), jnp.float32),
            pltpu.VMEM((_CHUNK, H), jnp.float32),
            pltpu.SemaphoreType.DMA,
        ],
    )(_gather_body)


_BN = 2000


def _dense_body(x_ref, hcat_ref, ccat_ref, tf_ref,
                wx_ref, wh_ref, wf_ref, biou_ref, bious_ref, bf_ref,
                h_out_ref, c_out_ref):
    x = x_ref[...]
    hcat = hcat_ref[...]
    iou_both = (
        jnp.dot(x, wx_ref[...], preferred_element_type=jnp.float32)
        + jnp.dot(hcat, wh_ref[...], preferred_element_type=jnp.float32)
    )
    f = jax.nn.sigmoid(
        jnp.dot(hcat, wf_ref[...], preferred_element_type=jnp.float32)
        + bf_ref[...]
    )
    ccat = ccat_ref[...]
    c1 = ccat[:, :H]
    c2 = ccat[:, H:]
    c_n = f[:, :H] * c1 + f[:, H:2 * H] * c2
    c_sm = f[:, 2 * H:3 * H] * c1 + f[:, 3 * H:] * c2
    iou_n = iou_both[:, :3 * H] + biou_ref[...]
    iou_sm = iou_both[:, 3 * H:] + bious_ref[...]
    tm = tf_ref[...]
    iou = iou_n + tm * (iou_sm - iou_n)
    c_r = c_n + tm * (c_sm - c_n)
    c_out = jax.nn.sigmoid(iou[:, :H]) * jnp.tanh(iou[:, 2 * H:]) + c_r
    c_out_ref[...] = c_out
    h_out_ref[...] = jax.nn.sigmoid(iou[:, H:2 * H]) * jnp.tanh(c_out)


def _dense_call(x, hcat, ccat, tf, wx, wh, wf, biou, bious, bf):
    grid = (N // _BN,)
    row_spec = lambda w: pl.BlockSpec((_BN, w), lambda i: (i, 0))
    full_spec = lambda a, b: pl.BlockSpec((a, b), lambda i: (0, 0))
    return pl.pallas_call(
        _dense_body,
        grid=grid,
        in_specs=[
            row_spec(X),
            row_spec(2 * H),
            row_spec(2 * H),
            row_spec(1),
            full_spec(X, 6 * H),
            full_spec(2 * H, 6 * H),
            full_spec(2 * H, 4 * H),
            full_spec(1, 3 * H),
            full_spec(1, 3 * H),
            full_spec(1, 4 * H),
        ],
        out_specs=[row_spec(H), row_spec(H)],
        out_shape=[
            jax.ShapeDtypeStruct((N, H), jnp.float32),
            jax.ShapeDtypeStruct((N, H), jnp.float32),
        ],
    )(x, hcat, ccat, tf, wx, wh, wf, biou, bious, bf)


def kernel(x, h_src, c_src, child_idx, t, W_iou, U_iou, b_iou, U_f_w, U_f_b,
           W_iou_s, U_iou_s, b_iou_s, U_f_s_w, U_f_s_b):
    idx = jnp.concatenate(
        [child_idx.reshape(-1),
         jnp.zeros((_PAD2N - 2 * N,), dtype=jnp.int32)])

    h_mail, c_mail = _make_sc_gather()(h_src, c_src, idx)
    hcat = h_mail.reshape(_PAD2N // 2, 2 * H)
    ccat = c_mail.reshape(_PAD2N // 2, 2 * H)

    wx = jnp.concatenate([W_iou, W_iou_s], axis=1)
    wh = jnp.concatenate(
        [U_iou, jnp.concatenate([U_iou_s, U_iou_s], axis=0)], axis=1)
    z = jnp.zeros((H, H), dtype=jnp.float32)
    ufs_bd = jnp.block([[U_f_s_w, z], [z, U_f_s_w]])
    wf = jnp.concatenate([U_f_w, ufs_bd], axis=1)
    bf = jnp.concatenate([U_f_b, U_f_s_b, U_f_s_b]).reshape(1, 4 * H)
    tf = (t == 1).astype(jnp.float32).reshape(N, 1)

    h_out, c_out = _dense_call(x, hcat, ccat, tf, wx, wh, wf,
                               b_iou, b_iou_s, bf)
    return (h_out, c_out)
